# baseline (speedup 1.0000x reference)
import jax
import jax.numpy as jnp
from jax import lax
from jax.experimental import pallas as pl
from jax.experimental.pallas import tpu as pltpu
from jax.experimental.pallas import tpu_sc as plsc

_BATCH = 4096
_L = 200
_D = 64
_LANES = 16
_GROUPS = _D // _LANES

_REPACK_COLS = 8192
_C_SAMPLES = 128


def _repack_body(t_ref, o_ref):
    x = t_ref[...]
    eye = jnp.eye(_D, dtype=jnp.float32)
    xt = lax.dot_general(x, eye, (((0,), (0,)), ((), ())),
                         precision=lax.Precision.HIGHEST,
                         preferred_element_type=jnp.float32)
    o_ref[...] = jnp.concatenate([xt, jnp.zeros_like(xt)], axis=1)


def _repack_table(table_t):
    v = table_t.shape[1]
    return pl.pallas_call(
        _repack_body,
        grid=(pl.cdiv(v, _REPACK_COLS),),
        in_specs=[pl.BlockSpec((_D, _REPACK_COLS), lambda i: (0, i))],
        out_specs=pl.BlockSpec((_REPACK_COLS, 2 * _D), lambda i: (i, 0)),
        out_shape=jax.ShapeDtypeStruct((v, 2 * _D), jnp.float32),
    )(table_t)


def _ln_body(x_ref, w_ref, b_ref, o_ref):
    x = x_ref[...]
    n = jnp.float32(1.0 / (_L * _D))
    m = jnp.sum(x, axis=(1, 2), keepdims=True) * n
    msq = jnp.sum(x * x, axis=(1, 2), keepdims=True) * n
    inv = lax.rsqrt(msq - m * m + jnp.float32(1e-5))
    y = (x - m) * inv * w_ref[...][None] + b_ref[...][None]
    o_ref[...] = jnp.transpose(y, (1, 2, 0))


def _layer_norm(gathered, w, b):
    return pl.pallas_call(
        _ln_body,
        grid=(_BATCH // _C_SAMPLES,),
        in_specs=[
            pl.BlockSpec((_C_SAMPLES, _L, _D), lambda i: (i, 0, 0)),
            pl.BlockSpec((_L, _D), lambda i: (0, 0)),
            pl.BlockSpec((_L, _D), lambda i: (0, 0)),
        ],
        out_specs=pl.BlockSpec((_L, _D, _C_SAMPLES), lambda i: (0, 0, i)),
        out_shape=jax.ShapeDtypeStruct((_L, _D, _BATCH), jnp.float32),
    )(gathered, w, b)


def _build_gather_kernel():
    info = plsc.get_sparse_core_info()
    nc, ns = info.num_cores, info.num_subcores
    nw = nc * ns
    per_w = _BATCH // nw

    mesh = plsc.VectorSubcoreMesh(core_axis_name="c", subcore_axis_name="s")

    @pl.kernel(
        mesh=mesh,
        out_type=jax.ShapeDtypeStruct((_BATCH, _L, _D), jnp.float32),
        scratch_types={
            "idx_raw": pltpu.VMEM((per_w * _L,), jnp.int32),
            "rows0": pltpu.VMEM((_L, 2 * _D), jnp.float32),
            "rows1": pltpu.VMEM((_L, 2 * _D), jnp.float32),
            "ob0": pltpu.VMEM((_L, _D), jnp.float32),
            "ob1": pltpu.VMEM((_L, _D), jnp.float32),
            "g0": pltpu.SemaphoreType.DMA,
            "g1": pltpu.SemaphoreType.DMA,
            "o0": pltpu.SemaphoreType.DMA,
            "o1": pltpu.SemaphoreType.DMA,
        },
        compiler_params=pltpu.CompilerParams(
            needs_layout_passes=False, use_tc_tiling_on_sc=True
        ),
    )
    def k(ids_hbm, table128_hbm, out_hbm, *, idx_raw, rows0, rows1,
          ob0, ob1, g0, g1, o0, o1):
        wid = lax.axis_index("s") * nc + lax.axis_index("c")
        base = wid * per_w

        pltpu.sync_copy(ids_hbm.at[pl.ds(base * _L, per_w * _L)], idx_raw)

        def g_start(s, buf, sem):
            pltpu.make_async_copy(
                table128_hbm.at[idx_raw.at[pl.ds(s * _L, _L)]], buf, sem
            ).start()

        def g_wait(buf, sem):
            pltpu.make_async_copy(
                table128_hbm.at[idx_raw.at[pl.ds(0, _L)]], buf, sem).wait()

        def o_start(ob, s, sem):
            pltpu.make_async_copy(ob, out_hbm.at[s], sem).start()

        def o_wait(ob, sem):
            pltpu.make_async_copy(ob, out_hbm.at[base], sem).wait()

        def compact(buf, ob):
            @plsc.parallel_loop(0, _L, 1, unroll=8)
            def cp(r):
                vs = [buf[r, pl.ds(g * _LANES, _LANES)]
                      for g in range(_GROUPS)]
                for g in range(_GROUPS):
                    ob[r, pl.ds(g * _LANES, _LANES)] = vs[g]

        g_start(0, rows0, g0)

        def body(j, _):
            s0 = 2 * j

            @pl.when(j != 0)
            def _():
                o_wait(ob1, o1)

            g_start(s0 + 1, rows1, g1)
            g_wait(rows0, g0)
            compact(rows0, ob0)
            o_start(ob0, base + s0, o0)
            g_wait(rows1, g1)
            compact(rows1, ob1)
            o_wait(ob0, o0)

            @pl.when(j != per_w // 2 - 1)
            def _():
                g_start(s0 + 2, rows0, g0)

            o_start(ob1, base + s0 + 1, o1)
            return 0

        lax.fori_loop(0, per_w // 2, body, 0)
        o_wait(ob1, o1)

    return k


_gather_call = None


def kernel(input_ids, table, ln_weight, ln_bias):
    global _gather_call
    if _gather_call is None:
        _gather_call = _build_gather_kernel()
    ids_flat = jnp.reshape(input_ids, (-1,))
    table128 = _repack_table(jnp.transpose(table))
    gathered = _gather_call(ids_flat, table128)
    out_t = _layer_norm(gathered, ln_weight, ln_bias)
    return jnp.transpose(out_t, (2, 0, 1))


# speedup vs baseline: 1.1891x; 1.1891x over previous
import jax
import jax.numpy as jnp
from jax import lax
from jax.experimental import pallas as pl
from jax.experimental.pallas import tpu as pltpu
from jax.experimental.pallas import tpu_sc as plsc

_BATCH = 4096
_L = 200
_D = 64
_LANES = 16
_GROUPS = _D // _LANES

_REPACK_COLS = 8192
_C_SAMPLES = 128


def _repack_body(t_ref, o_ref):
    xt = jnp.transpose(t_ref[...])
    o_ref[...] = jnp.concatenate([xt, jnp.zeros_like(xt)], axis=1)


def _repack_table(table_t):
    v = table_t.shape[1]
    return pl.pallas_call(
        _repack_body,
        grid=(pl.cdiv(v, _REPACK_COLS),),
        in_specs=[pl.BlockSpec((_D, _REPACK_COLS), lambda i: (0, i))],
        out_specs=pl.BlockSpec((_REPACK_COLS, 2 * _D), lambda i: (i, 0)),
        out_shape=jax.ShapeDtypeStruct((v, 2 * _D), jnp.float32),
    )(table_t)


def _ln_body(x_ref, w_ref, b_ref, o_ref):
    x = x_ref[...]
    n = jnp.float32(1.0 / (_L * _D))
    m = jnp.sum(x, axis=(1, 2), keepdims=True) * n
    msq = jnp.sum(x * x, axis=(1, 2), keepdims=True) * n
    inv = lax.rsqrt(msq - m * m + jnp.float32(1e-5))
    y = (x - m) * inv * w_ref[...][None] + b_ref[...][None]
    o_ref[...] = jnp.transpose(y, (1, 2, 0))


def _layer_norm(gathered, w, b):
    return pl.pallas_call(
        _ln_body,
        grid=(_BATCH // _C_SAMPLES,),
        in_specs=[
            pl.BlockSpec((_C_SAMPLES, _L, _D), lambda i: (i, 0, 0)),
            pl.BlockSpec((_L, _D), lambda i: (0, 0)),
            pl.BlockSpec((_L, _D), lambda i: (0, 0)),
        ],
        out_specs=pl.BlockSpec((_L, _D, _C_SAMPLES), lambda i: (0, 0, i)),
        out_shape=jax.ShapeDtypeStruct((_L, _D, _BATCH), jnp.float32),
    )(gathered, w, b)


def _build_gather_kernel():
    info = plsc.get_sparse_core_info()
    nc, ns = info.num_cores, info.num_subcores
    nw = nc * ns
    per_w = _BATCH // nw

    mesh = plsc.VectorSubcoreMesh(core_axis_name="c", subcore_axis_name="s")

    @pl.kernel(
        mesh=mesh,
        out_type=jax.ShapeDtypeStruct((_BATCH, _L, _D), jnp.float32),
        scratch_types={
            "idx_raw": pltpu.VMEM((per_w * _L,), jnp.int32),
            "rows0": pltpu.VMEM((_L, 2 * _D), jnp.float32),
            "rows1": pltpu.VMEM((_L, 2 * _D), jnp.float32),
            "ob0": pltpu.VMEM((_L, _D), jnp.float32),
            "ob1": pltpu.VMEM((_L, _D), jnp.float32),
            "g0": pltpu.SemaphoreType.DMA,
            "g1": pltpu.SemaphoreType.DMA,
            "o0": pltpu.SemaphoreType.DMA,
            "o1": pltpu.SemaphoreType.DMA,
        },
        compiler_params=pltpu.CompilerParams(
            needs_layout_passes=False, use_tc_tiling_on_sc=True
        ),
    )
    def k(ids_hbm, table128_hbm, out_hbm, *, idx_raw, rows0, rows1,
          ob0, ob1, g0, g1, o0, o1):
        wid = lax.axis_index("s") * nc + lax.axis_index("c")
        base = wid * per_w

        pltpu.sync_copy(ids_hbm.at[pl.ds(base * _L, per_w * _L)], idx_raw)

        def g_start(s, buf, sem):
            pltpu.make_async_copy(
                table128_hbm.at[idx_raw.at[pl.ds(s * _L, _L)]], buf, sem
            ).start()

        def g_wait(buf, sem):
            pltpu.make_async_copy(
                table128_hbm.at[idx_raw.at[pl.ds(0, _L)]], buf, sem).wait()

        def o_start(ob, s, sem):
            pltpu.make_async_copy(ob, out_hbm.at[s], sem).start()

        def o_wait(ob, sem):
            pltpu.make_async_copy(ob, out_hbm.at[base], sem).wait()

        def compact(buf, ob):
            @plsc.parallel_loop(0, _L, 1, unroll=8)
            def cp(r):
                vs = [buf[r, pl.ds(g * _LANES, _LANES)]
                      for g in range(_GROUPS)]
                for g in range(_GROUPS):
                    ob[r, pl.ds(g * _LANES, _LANES)] = vs[g]

        g_start(0, rows0, g0)

        def body(j, _):
            s0 = 2 * j

            @pl.when(j != 0)
            def _():
                o_wait(ob1, o1)

            g_start(s0 + 1, rows1, g1)
            g_wait(rows0, g0)
            compact(rows0, ob0)
            o_start(ob0, base + s0, o0)
            g_wait(rows1, g1)
            compact(rows1, ob1)
            o_wait(ob0, o0)

            @pl.when(j != per_w // 2 - 1)
            def _():
                g_start(s0 + 2, rows0, g0)

            o_start(ob1, base + s0 + 1, o1)
            return 0

        lax.fori_loop(0, per_w // 2, body, 0)
        o_wait(ob1, o1)

    return k


_gather_call = None


def kernel(input_ids, table, ln_weight, ln_bias):
    global _gather_call
    if _gather_call is None:
        _gather_call = _build_gather_kernel()
    ids_flat = jnp.reshape(input_ids, (-1,))
    table128 = _repack_table(jnp.transpose(table))
    gathered = _gather_call(ids_flat, table128)
    out_t = _layer_norm(gathered, ln_weight, ln_bias)
    return jnp.transpose(out_t, (2, 0, 1))
